# baseline (speedup 1.0000x reference)
import functools

import jax
import jax.numpy as jnp
from jax import lax
from jax.experimental import pallas as pl
from jax.experimental.pallas import tpu as pltpu
from jax.experimental.pallas import tpu_sc as plsc

_DIM = 64
_NW = 32
_WINDOW = 128
_SLAB = 512
_CBLK = 2048


@functools.partial(jax.jit, static_argnums=(2,))
def _gather_pairs(pairs, widx, num_idx):
    mesh = plsc.VectorSubcoreMesh(core_axis_name="core",
                                  subcore_axis_name="subcore")
    ipw = num_idx // _NW
    nchunk = ipw // _WINDOW
    cps = _SLAB // _WINDOW
    nround = nchunk // cps

    @functools.partial(
        pl.kernel,
        out_type=jax.ShapeDtypeStruct((num_idx, 2 * _DIM), jnp.float32),
        mesh=mesh,
        scratch_types=[
            pltpu.VMEM((nchunk, _WINDOW), jnp.int32),
            pltpu.VMEM((_SLAB, 2 * _DIM), jnp.float32),
            pltpu.SemaphoreType.DMA,
        ],
    )
    def gather_kernel(x_hbm, i_hbm, o_hbm, idx_v, rows_v, sem):
        wid = lax.axis_index("subcore") * 2 + lax.axis_index("core")
        pltpu.sync_copy(i_hbm.at[wid], idx_v)
        for r in range(nround):
            copies = []
            for j in range(cps):
                copies.append(pltpu.async_copy(
                    x_hbm.at[idx_v.at[r * cps + j]],
                    rows_v.at[pl.ds(j * _WINDOW, _WINDOW)],
                    sem))
            for c in copies:
                c.wait()
            pltpu.sync_copy(
                rows_v, o_hbm.at[pl.ds(wid * ipw + r * _SLAB, _SLAB)])

    return gather_kernel(pairs, widx)


@functools.partial(jax.jit, static_argnums=(2,))
def _select_concat(rows, hi, b):
    grid = b // _CBLK
    off = b // _CBLK

    def body(a_ref, s_ref, ha_ref, hs_ref, o_ref):
        a = a_ref[...]
        s = s_ref[...]
        av = jnp.where(ha_ref[...] == 1, a[:, _DIM:], a[:, :_DIM])
        sv = jnp.where(hs_ref[...] == 1, s[:, _DIM:], s[:, :_DIM])
        o_ref[...] = jnp.concatenate([av, sv], axis=1)

    return pl.pallas_call(
        body,
        grid=(grid,),
        in_specs=[pl.BlockSpec((_CBLK, 2 * _DIM), lambda i: (i, 0)),
                  pl.BlockSpec((_CBLK, 2 * _DIM), lambda i: (off + i, 0)),
                  pl.BlockSpec((_CBLK, 1), lambda i: (i, 0)),
                  pl.BlockSpec((_CBLK, 1), lambda i: (off + i, 0))],
        out_specs=pl.BlockSpec((_CBLK, 2 * _DIM), lambda i: (i, 0)),
        out_shape=jax.ShapeDtypeStruct((b, 2 * _DIM), jnp.float32),
    )(rows, rows, hi, hi)


def kernel(obj_category, sub_category, word_embs):
    b = obj_category.shape[0]
    vocab = word_embs.shape[0]
    idx = jnp.concatenate(
        [obj_category.astype(jnp.int32), sub_category.astype(jnp.int32)])
    pairs = word_embs.reshape(vocab // 2, 2 * _DIM)
    widx = (idx >> 1).reshape(_NW, (2 * b) // (_NW * _WINDOW), _WINDOW)
    hi = (idx & 1).reshape(2 * b, 1)
    gathered = _gather_pairs(pairs, widx, 2 * b)
    return _select_concat(gathered, hi, b)


# speedup vs baseline: 1.2579x; 1.2579x over previous
import functools

import jax
import jax.numpy as jnp
from jax import lax
from jax.experimental import pallas as pl
from jax.experimental.pallas import tpu as pltpu
from jax.experimental.pallas import tpu_sc as plsc

_DIM = 64
_NW = 32
_WINDOW = 128
_SLAB = 512


_TBLK = 1024


@jax.jit
def _transpose_pairs(tbl_t):
    vocab = tbl_t.shape[1]
    grid = (vocab + 2 * _TBLK - 1) // (2 * _TBLK)
    last = (vocab - 1) // _TBLK

    def body(x1_ref, x2_ref, o_ref):
        o_ref[...] = jnp.concatenate([x1_ref[...].T, x2_ref[...].T], axis=1)

    return pl.pallas_call(
        body,
        grid=(grid,),
        in_specs=[pl.BlockSpec((_DIM, _TBLK),
                               lambda i: (0, jnp.minimum(2 * i, last))),
                  pl.BlockSpec((_DIM, _TBLK),
                               lambda i: (0, jnp.minimum(2 * i + 1, last)))],
        out_specs=pl.BlockSpec((_TBLK, 2 * _DIM), lambda i: (i, 0)),
        out_shape=jax.ShapeDtypeStruct((grid * _TBLK, 2 * _DIM), jnp.float32),
    )(tbl_t, tbl_t)


@functools.partial(jax.jit, static_argnums=(2,))
def _gather_pairs(pairs, pidx, num_idx):
    mesh = plsc.VectorSubcoreMesh(core_axis_name="core",
                                  subcore_axis_name="subcore")
    ipw = num_idx // _NW
    nchunk = ipw // _WINDOW
    cps = _SLAB // _WINDOW
    nround = nchunk // cps

    @functools.partial(
        pl.kernel,
        out_type=jax.ShapeDtypeStruct((num_idx, 2 * _DIM), jnp.float32),
        mesh=mesh,
        scratch_types=[
            pltpu.VMEM((nchunk, _WINDOW), jnp.int32),
            pltpu.VMEM((_SLAB, 2 * _DIM), jnp.float32),
            pltpu.SemaphoreType.DMA,
        ],
    )
    def gather_kernel(x_hbm, i_hbm, o_hbm, idx_v, rows_v, sem):
        wid = lax.axis_index("subcore") * 2 + lax.axis_index("core")
        pltpu.sync_copy(i_hbm.at[wid], idx_v)
        for r in range(nround):
            copies = []
            for j in range(cps):
                copies.append(pltpu.async_copy(
                    x_hbm.at[idx_v.at[r * cps + j]],
                    rows_v.at[pl.ds(j * _WINDOW, _WINDOW)],
                    sem))
            for c in copies:
                c.wait()
            pltpu.sync_copy(
                rows_v, o_hbm.at[pl.ds(wid * ipw + r * _SLAB, _SLAB)])

    return gather_kernel(pairs, pidx)


def kernel(obj_category, sub_category, word_embs):
    b = obj_category.shape[0]
    idx = jnp.stack(
        [obj_category.astype(jnp.int32), sub_category.astype(jnp.int32)],
        axis=1,
    ).reshape(2 * b)
    pairs = _transpose_pairs(word_embs.T)
    pidx = ((idx >> 11) << 10) | (idx & (_TBLK - 1))
    pidx = pidx.reshape(_NW, (2 * b) // (_NW * _WINDOW), _WINDOW)
    gathered = _gather_pairs(pairs, pidx, 2 * b)
    hi = ((idx >> 10) & 1)[:, None] == 1
    half = jnp.where(hi, gathered[:, _DIM:], gathered[:, :_DIM])
    return half.reshape(b, 2 * _DIM)
